# speedup vs baseline: 5.0062x; 5.0062x over previous
import functools

import jax
import jax.numpy as jnp
from jax.experimental import pallas as pl
from jax.experimental.pallas import tpu as pltpu


def _fused_body(x_ref, we_ref, wdt_ref, benc_ref, bdec_ref, out_ref,
                slab, thr, acc, *, TB, FB, NF, K):
    p = pl.program_id(1)
    f = pl.program_id(2)

    @pl.when(p == 0)
    def _encode():
        xb = x_ref[...] - bdec_ref[...]
        pre = jax.lax.dot_general(
            xb, we_ref[...], (((1,), (1,)), ((), ())),
            preferred_element_type=jnp.float32)
        pre = pre + benc_ref[...]
        off = pl.multiple_of(f * FB, FB)
        slab[:, pl.ds(off, FB)] = jnp.maximum(pre, 0.0)

    @pl.when((p == 0) & (f == NF - 1))
    def _select():
        s = slab[...]
        g0 = jnp.max(s, axis=1, keepdims=True)

        def step(i, g):
            return jnp.max(jnp.where(s < g, s, -1.0), axis=1, keepdims=True)

        gK = jax.lax.fori_loop(0, K - 1, step, g0)
        thr[...] = jnp.maximum(gK, jnp.float32(1e-37))

    @pl.when(p == 1)
    def _decode():
        off = pl.multiple_of(f * FB, FB)
        sb = slab[:, pl.ds(off, FB)]
        enc = jnp.where(sb >= thr[...], sb, 0.0)
        part = jax.lax.dot_general(
            enc, wdt_ref[...], (((1,), (0,)), ((), ())),
            preferred_element_type=jnp.float32)

        @pl.when(f == 0)
        def _():
            acc[...] = part

        @pl.when(f > 0)
        def _():
            acc[...] = acc[...] + part

        @pl.when(f == NF - 1)
        def _():
            out_ref[...] = acc[...] + bdec_ref[...]


def kernel(x, W_enc, b_enc, W_dec, b_dec):
    N, D = x.shape
    F = W_enc.shape[0]
    K = 64
    TB = min(256, N)
    FB = min(512, F)
    NF = F // FB
    W_dec_T = W_dec.T
    b_enc2 = b_enc.reshape(1, F)
    b_dec2 = b_dec.reshape(1, D)

    grid = (N // TB, 2, NF)
    body = functools.partial(_fused_body, TB=TB, FB=FB, NF=NF, K=K)
    out = pl.pallas_call(
        body,
        grid=grid,
        in_specs=[
            pl.BlockSpec((TB, D), lambda t, p, f: (t, 0)),
            pl.BlockSpec((FB, D), lambda t, p, f: (jnp.where(p == 0, f, NF - 1), 0)),
            pl.BlockSpec((FB, D), lambda t, p, f: (jnp.where(p == 1, f, 0), 0)),
            pl.BlockSpec((1, FB), lambda t, p, f: (0, jnp.where(p == 0, f, NF - 1))),
            pl.BlockSpec((1, D), lambda t, p, f: (0, 0)),
        ],
        out_specs=pl.BlockSpec((TB, D), lambda t, p, f: (t, 0)),
        out_shape=jax.ShapeDtypeStruct((N, D), jnp.float32),
        scratch_shapes=[
            pltpu.VMEM((TB, F), jnp.float32),
            pltpu.VMEM((TB, 1), jnp.float32),
            pltpu.VMEM((TB, D), jnp.float32),
        ],
        compiler_params=pltpu.CompilerParams(
            dimension_semantics=("arbitrary", "arbitrary", "arbitrary"),
        ),
    )(x, W_enc, W_dec_T, b_enc2, b_dec2)
    return out


# speedup vs baseline: 8.1324x; 1.6245x over previous
import functools

import jax
import jax.numpy as jnp
from jax.experimental import pallas as pl
from jax.experimental.pallas import tpu as pltpu


def _fused_body(x_ref, we_ref, wdt_ref, benc_ref, bdec_ref, out_ref,
                slab, thr, acc, *, TB, FB, NF, K):
    p = pl.program_id(1)
    f = pl.program_id(2)

    @pl.when(p == 0)
    def _encode():
        xb = x_ref[...] - bdec_ref[...]
        pre = jax.lax.dot_general(
            xb, we_ref[...], (((1,), (1,)), ((), ())),
            preferred_element_type=jnp.float32)
        pre = pre + benc_ref[...]
        off = pl.multiple_of(f * FB, FB)
        slab[:, pl.ds(off, FB)] = jnp.maximum(pre, 0.0)

    @pl.when((p == 0) & (f == NF - 1))
    def _select():
        si = jax.lax.bitcast_convert_type(slab[...], jnp.int32)
        lo0 = jnp.zeros((si.shape[0], 1), jnp.int32)
        hi0 = jnp.full((si.shape[0], 1), jnp.int32(0x7F800000))

        def step(i, carry):
            lo, hi = carry
            mid = jax.lax.shift_right_logical(lo + hi, 1)
            c = jnp.sum((si >= mid).astype(jnp.int32), axis=1, keepdims=True)
            big = c >= K
            return jnp.where(big, mid, lo), jnp.where(big, hi, mid)

        lo, hi = jax.lax.fori_loop(0, 31, step, (lo0, hi0))
        t = jax.lax.bitcast_convert_type(lo, jnp.float32)
        thr[...] = jnp.maximum(t, jnp.float32(1e-37))

    @pl.when(p == 1)
    def _decode():
        off = pl.multiple_of(f * FB, FB)
        sb = slab[:, pl.ds(off, FB)]
        enc = jnp.where(sb >= thr[...], sb, 0.0)
        part = jax.lax.dot_general(
            enc, wdt_ref[...], (((1,), (0,)), ((), ())),
            preferred_element_type=jnp.float32)

        @pl.when(f == 0)
        def _():
            acc[...] = part

        @pl.when(f > 0)
        def _():
            acc[...] = acc[...] + part

        @pl.when(f == NF - 1)
        def _():
            out_ref[...] = acc[...] + bdec_ref[...]


def kernel(x, W_enc, b_enc, W_dec, b_dec):
    N, D = x.shape
    F = W_enc.shape[0]
    K = 64
    TB = min(256, N)
    FB = min(512, F)
    NF = F // FB
    W_dec_T = W_dec.T
    b_enc2 = b_enc.reshape(1, F)
    b_dec2 = b_dec.reshape(1, D)

    grid = (N // TB, 2, NF)
    body = functools.partial(_fused_body, TB=TB, FB=FB, NF=NF, K=K)
    out = pl.pallas_call(
        body,
        grid=grid,
        in_specs=[
            pl.BlockSpec((TB, D), lambda t, p, f: (t, 0)),
            pl.BlockSpec((FB, D), lambda t, p, f: (jnp.where(p == 0, f, NF - 1), 0)),
            pl.BlockSpec((FB, D), lambda t, p, f: (jnp.where(p == 1, f, 0), 0)),
            pl.BlockSpec((1, FB), lambda t, p, f: (0, jnp.where(p == 0, f, NF - 1))),
            pl.BlockSpec((1, D), lambda t, p, f: (0, 0)),
        ],
        out_specs=pl.BlockSpec((TB, D), lambda t, p, f: (t, 0)),
        out_shape=jax.ShapeDtypeStruct((N, D), jnp.float32),
        scratch_shapes=[
            pltpu.VMEM((TB, F), jnp.float32),
            pltpu.VMEM((TB, 1), jnp.float32),
            pltpu.VMEM((TB, D), jnp.float32),
        ],
        compiler_params=pltpu.CompilerParams(
            dimension_semantics=("arbitrary", "arbitrary", "arbitrary"),
        ),
    )(x, W_enc, W_dec_T, b_enc2, b_dec2)
    return out
